# baseline (speedup 1.0000x reference)
import functools

import numpy as np
import jax
import jax.numpy as jnp
from jax import lax
from jax.experimental import pallas as pl
from jax.experimental.pallas import tpu as pltpu
from jax.experimental.pallas import tpu_sc as plsc

SLOWFAST_ALPHA = 4

NC = 2
NS = 16


def kernel(frames):
    C, T, H, W = frames.shape
    n = T // SLOWFAST_ALPHA
    idx = [int(v) for v in np.trunc(np.linspace(0.0, T - 1, n)).astype(np.int64)]
    HW = H * W
    R = C * T
    NW = NC * NS
    RPW = R // NW
    x2 = frames.reshape(R, HW)

    mesh = plsc.VectorSubcoreMesh(
        core_axis_name="c", subcore_axis_name="s", num_cores=NC, num_subcores=NS
    )

    @functools.partial(
        pl.kernel,
        out_type=[
            jax.ShapeDtypeStruct((C * n, HW), frames.dtype),
            jax.ShapeDtypeStruct((R, HW), frames.dtype),
        ],
        mesh=mesh,
        scratch_types=[
            pltpu.VMEM((HW,), jnp.float32),
            pltpu.VMEM((HW,), jnp.float32),
            pltpu.SemaphoreType.DMA((2,)),
            pltpu.SemaphoreType.DMA((2,)),
            pltpu.SemaphoreType.DMA,
        ],
    )
    def run(x_hbm, slow_hbm, fast_hbm, buf0, buf1, in_sems, out_sems, slow_sem):
        wid = lax.axis_index("s") * NC + lax.axis_index("c")
        base = wid * RPW
        bufs = (buf0, buf1)

        def row_info(k):
            r = base + k
            c = r // T
            t = r - c * T
            sel = t == idx[0]
            j = jnp.int32(0)
            for v in idx[1:]:
                sel = sel | (t == v)
            for v in idx:
                j = j + (t > v).astype(jnp.int32)
            return r, sel, c * n + j

        def start_in(k):
            r = base + k
            pltpu.make_async_copy(x_hbm.at[r], bufs[k % 2], in_sems.at[k % 2]).start()

        def wait_in(k):
            r = base + k
            pltpu.make_async_copy(x_hbm.at[r], bufs[k % 2], in_sems.at[k % 2]).wait()

        def start_outs(k):
            r, sel, slow_row = row_info(k)
            pltpu.make_async_copy(bufs[k % 2], fast_hbm.at[r], out_sems.at[k % 2]).start()

            @pl.when(sel)
            def _():
                pltpu.make_async_copy(bufs[k % 2], slow_hbm.at[slow_row], slow_sem).start()

        def wait_outs(k):
            r, sel, slow_row = row_info(k)
            pltpu.make_async_copy(bufs[k % 2], fast_hbm.at[r], out_sems.at[k % 2]).wait()

            @pl.when(sel)
            def _():
                pltpu.make_async_copy(bufs[k % 2], slow_hbm.at[slow_row], slow_sem).wait()

        start_in(0)
        if RPW > 1:
            start_in(1)
        wait_in(0)
        start_outs(0)
        for k in range(1, RPW):
            wait_in(k)
            wait_outs(k - 1)
            if k + 1 < RPW:
                start_in(k + 1)
            start_outs(k)
        wait_outs(RPW - 1)

    slow, fast = run(x2)
    return (slow.reshape(C, n, H, W), fast.reshape(C, T, H, W))


# speedup vs baseline: 4.2787x; 4.2787x over previous
import numpy as np
import jax
import jax.numpy as jnp
from jax.experimental import pallas as pl

SLOWFAST_ALPHA = 4


def kernel(frames):
    C, T, H, W = frames.shape
    n = T // SLOWFAST_ALPHA
    idx = [int(v) for v in np.trunc(np.linspace(0.0, T - 1, n)).astype(np.int64)]

    NB = 4
    TB = T // NB
    SB = n // NB
    locals_per_chunk = []
    for b in range(NB):
        loc = [t - b * TB for t in idx if b * TB <= t < (b + 1) * TB]
        assert len(loc) == SB, (b, loc)
        locals_per_chunk.append(loc)

    def body(x_ref, slow_ref, fast_ref):
        tb = pl.program_id(1)
        fast_ref[...] = x_ref[...]
        for b in range(NB):
            @pl.when(tb == b)
            def _(b=b):
                for j, loc in enumerate(locals_per_chunk[b]):
                    slow_ref[0, j] = x_ref[0, loc]

    return pl.pallas_call(
        body,
        grid=(C, NB),
        in_specs=[pl.BlockSpec((1, TB, H, W), lambda c, tb: (c, tb, 0, 0))],
        out_specs=[
            pl.BlockSpec((1, SB, H, W), lambda c, tb: (c, tb, 0, 0)),
            pl.BlockSpec((1, TB, H, W), lambda c, tb: (c, tb, 0, 0)),
        ],
        out_shape=[
            jax.ShapeDtypeStruct((C, n, H, W), frames.dtype),
            jax.ShapeDtypeStruct((C, T, H, W), frames.dtype),
        ],
    )(frames)


# speedup vs baseline: 4.7028x; 1.0991x over previous
import numpy as np
import jax
import jax.numpy as jnp
from jax.experimental import pallas as pl

SLOWFAST_ALPHA = 4


def kernel(frames):
    C, T, H, W = frames.shape
    n = T // SLOWFAST_ALPHA
    idx = [int(v) for v in np.trunc(np.linspace(0.0, T - 1, n)).astype(np.int64)]

    NB = 2
    TB = T // NB
    SB = n // NB
    locals_per_chunk = []
    for b in range(NB):
        loc = [t - b * TB for t in idx if b * TB <= t < (b + 1) * TB]
        assert len(loc) == SB, (b, loc)
        locals_per_chunk.append(loc)

    def body(x_ref, slow_ref, fast_ref):
        tb = pl.program_id(1)
        fast_ref[...] = x_ref[...]
        for b in range(NB):
            @pl.when(tb == b)
            def _(b=b):
                for j, loc in enumerate(locals_per_chunk[b]):
                    slow_ref[0, j] = x_ref[0, loc]

    return pl.pallas_call(
        body,
        grid=(C, NB),
        in_specs=[pl.BlockSpec((1, TB, H, W), lambda c, tb: (c, tb, 0, 0))],
        out_specs=[
            pl.BlockSpec((1, SB, H, W), lambda c, tb: (c, tb, 0, 0)),
            pl.BlockSpec((1, TB, H, W), lambda c, tb: (c, tb, 0, 0)),
        ],
        out_shape=[
            jax.ShapeDtypeStruct((C, n, H, W), frames.dtype),
            jax.ShapeDtypeStruct((C, T, H, W), frames.dtype),
        ],
    )(frames)


# speedup vs baseline: 5.4229x; 1.1531x over previous
import numpy as np
import jax
import jax.numpy as jnp
from jax.experimental import pallas as pl

SLOWFAST_ALPHA = 4


def kernel(frames):
    C, T, H, W = frames.shape
    n = T // SLOWFAST_ALPHA
    idx = [int(v) for v in np.trunc(np.linspace(0.0, T - 1, n)).astype(np.int64)]

    NB = 1
    TB = T // NB
    SB = n // NB
    locals_per_chunk = []
    for b in range(NB):
        loc = [t - b * TB for t in idx if b * TB <= t < (b + 1) * TB]
        assert len(loc) == SB, (b, loc)
        locals_per_chunk.append(loc)

    def body(x_ref, slow_ref, fast_ref):
        tb = pl.program_id(1)
        fast_ref[...] = x_ref[...]
        for b in range(NB):
            @pl.when(tb == b)
            def _(b=b):
                for j, loc in enumerate(locals_per_chunk[b]):
                    slow_ref[0, j] = x_ref[0, loc]

    return pl.pallas_call(
        body,
        grid=(C, NB),
        in_specs=[pl.BlockSpec((1, TB, H, W), lambda c, tb: (c, tb, 0, 0))],
        out_specs=[
            pl.BlockSpec((1, SB, H, W), lambda c, tb: (c, tb, 0, 0)),
            pl.BlockSpec((1, TB, H, W), lambda c, tb: (c, tb, 0, 0)),
        ],
        out_shape=[
            jax.ShapeDtypeStruct((C, n, H, W), frames.dtype),
            jax.ShapeDtypeStruct((C, T, H, W), frames.dtype),
        ],
    )(frames)
